# baseline (speedup 1.0000x reference)
import functools

import jax
import jax.numpy as jnp
from jax import lax
from jax.experimental import pallas as pl
from jax.experimental.pallas import tpu as pltpu
from jax.experimental.pallas import tpu_sc as plsc

_CHUNK_ROWS = 512
_NBUF = 3


def kernel(x, pos_table):
    seqlen = x.shape[1]
    _, dim = pos_table.shape

    info = plsc.get_sparse_core_info()
    nc, ns = info.num_cores, info.num_subcores
    assert seqlen % nc == 0
    rows_per_sc = seqlen // nc
    ch = min(_CHUNK_ROWS, rows_per_sc)
    assert rows_per_sc % ch == 0
    nch = rows_per_sc // ch
    nbuf = min(_NBUF, nch)

    mesh = plsc.VectorSubcoreMesh(core_axis_name="c", subcore_axis_name="s")

    @functools.partial(
        pl.kernel,
        mesh=mesh,
        out_type=jax.ShapeDtypeStruct((seqlen, dim), pos_table.dtype),
        scratch_types=(
            [pltpu.VMEM_SHARED((ch, dim), pos_table.dtype)] * nbuf
            + [pltpu.SemaphoreType.DMA] * (2 * nbuf)
        ),
    )
    def copy_k(table_hbm, out_hbm, *scratch):
        bufs = scratch[:nbuf]
        isems = scratch[nbuf : 2 * nbuf]
        osems = scratch[2 * nbuf :]
        sid = lax.axis_index("s")
        cid = lax.axis_index("c")

        @pl.when(sid == 0)
        def _():
            base = cid * rows_per_sc
            in_h = [None] * nch
            out_h = [None] * nch
            for i in range(nbuf):
                in_h[i] = pltpu.async_copy(
                    table_hbm.at[pl.ds(base + i * ch, ch)], bufs[i], isems[i]
                )
            for i in range(nch):
                b = i % nbuf
                in_h[i].wait()
                out_h[i] = pltpu.async_copy(
                    bufs[b], out_hbm.at[pl.ds(base + i * ch, ch)], osems[b]
                )
                if i + nbuf < nch:
                    out_h[i].wait()
                    in_h[i + nbuf] = pltpu.async_copy(
                        table_hbm.at[pl.ds(base + (i + nbuf) * ch, ch)],
                        bufs[b],
                        isems[b],
                    )
            for i in range(max(0, nch - nbuf), nch):
                out_h[i].wait()

    return copy_k(pos_table)


# speedup vs baseline: 1.0773x; 1.0773x over previous
import functools

import jax
import jax.numpy as jnp
from jax import lax
from jax.experimental import pallas as pl
from jax.experimental.pallas import tpu as pltpu
from jax.experimental.pallas import tpu_sc as plsc

_CHUNK_ROWS = 128
_NBUF = 3
_NDRIVERS = 4


def kernel(x, pos_table):
    seqlen = x.shape[1]
    _, dim = pos_table.shape

    info = plsc.get_sparse_core_info()
    nc, ns = info.num_cores, info.num_subcores
    nd = _NDRIVERS
    assert seqlen % (nc * nd) == 0
    rows_per_w = seqlen // (nc * nd)
    ch = min(_CHUNK_ROWS, rows_per_w)
    assert rows_per_w % ch == 0
    nch = rows_per_w // ch
    nbuf = min(_NBUF, nch)

    mesh = plsc.VectorSubcoreMesh(core_axis_name="c", subcore_axis_name="s")

    @functools.partial(
        pl.kernel,
        mesh=mesh,
        out_type=jax.ShapeDtypeStruct((seqlen, dim), pos_table.dtype),
        scratch_types=(
            [pltpu.VMEM_SHARED((nd, ch, dim), pos_table.dtype)] * nbuf
            + [pltpu.SemaphoreType.DMA] * (2 * nbuf)
        ),
    )
    def copy_k(table_hbm, out_hbm, *scratch):
        bufs = scratch[:nbuf]
        isems = scratch[nbuf : 2 * nbuf]
        osems = scratch[2 * nbuf :]
        sid = lax.axis_index("s")
        cid = lax.axis_index("c")

        @pl.when(sid < nd)
        def _():
            wid = cid * nd + sid
            base = wid * rows_per_w
            in_h = [None] * nch
            out_h = [None] * nch
            for i in range(nbuf):
                in_h[i] = pltpu.async_copy(
                    table_hbm.at[pl.ds(base + i * ch, ch)], bufs[i].at[sid], isems[i]
                )
            for i in range(nch):
                b = i % nbuf
                in_h[i].wait()
                out_h[i] = pltpu.async_copy(
                    bufs[b].at[sid], out_hbm.at[pl.ds(base + i * ch, ch)], osems[b]
                )
                if i + nbuf < nch:
                    out_h[i].wait()
                    in_h[i + nbuf] = pltpu.async_copy(
                        table_hbm.at[pl.ds(base + (i + nbuf) * ch, ch)],
                        bufs[b].at[sid],
                        isems[b],
                    )
            for i in range(max(0, nch - nbuf), nch):
                out_h[i].wait()

    return copy_k(pos_table)


# speedup vs baseline: 1.0791x; 1.0016x over previous
import functools

import jax
import jax.numpy as jnp
from jax import lax
from jax.experimental import pallas as pl
from jax.experimental.pallas import tpu as pltpu
from jax.experimental.pallas import tpu_sc as plsc

_ND = 8
_CH = 64
_NBUF = 3


def _ring_copy(table_hbm, out_hbm, base, rows, ch, bufs, isems, osems):
    nch = rows // ch
    nbuf = min(len(bufs), nch)
    in_h = [None] * nch
    out_h = [None] * nch
    for i in range(nbuf):
        in_h[i] = pltpu.async_copy(
            table_hbm.at[pl.ds(base + i * ch, ch)], bufs[i], isems[i]
        )
    for i in range(nch):
        b = i % nbuf
        in_h[i].wait()
        out_h[i] = pltpu.async_copy(
            bufs[b], out_hbm.at[pl.ds(base + i * ch, ch)], osems[b]
        )
        if i + nbuf < nch:
            out_h[i].wait()
            in_h[i + nbuf] = pltpu.async_copy(
                table_hbm.at[pl.ds(base + (i + nbuf) * ch, ch)], bufs[b], isems[b]
            )
    for i in range(max(0, nch - nbuf), nch):
        out_h[i].wait()


def kernel(x, pos_table):
    seqlen = x.shape[1]
    _, dim = pos_table.shape

    info = plsc.get_sparse_core_info()
    nc, ns = info.num_cores, info.num_subcores
    nw = nc * _ND
    assert seqlen % nw == 0
    rows_per_w = seqlen // nw
    assert rows_per_w % _CH == 0

    mesh = plsc.VectorSubcoreMesh(core_axis_name="c", subcore_axis_name="s")

    @functools.partial(
        pl.kernel,
        mesh=mesh,
        out_type=jax.ShapeDtypeStruct((seqlen, dim), pos_table.dtype),
        scratch_types=(
            [pltpu.VMEM_SHARED((_ND, _CH, dim), pos_table.dtype)] * _NBUF
            + [pltpu.SemaphoreType.DMA] * (2 * _NBUF)
        ),
    )
    def copy_k(table_hbm, out_hbm, *scratch):
        sbufs = scratch[:_NBUF]
        isems = scratch[_NBUF : 2 * _NBUF]
        osems = scratch[2 * _NBUF :]
        sid = lax.axis_index("s")
        cid = lax.axis_index("c")

        @pl.when(sid < _ND)
        def _():
            base = (cid * _ND + sid) * rows_per_w
            _ring_copy(
                table_hbm, out_hbm, base, rows_per_w, _CH,
                [b.at[sid] for b in sbufs], isems, osems,
            )

    return copy_k(pos_table)


# speedup vs baseline: 1.1217x; 1.0395x over previous
import functools

import jax
import jax.numpy as jnp
from jax import lax
from jax.experimental import pallas as pl
from jax.experimental.pallas import tpu as pltpu
from jax.experimental.pallas import tpu_sc as plsc

_CHUNK_ROWS = 32
_NBUF = 3


def kernel(x, pos_table):
    seqlen = x.shape[1]
    _, dim = pos_table.shape

    info = plsc.get_sparse_core_info()
    nc, ns = info.num_cores, info.num_subcores
    nw = nc * ns
    assert seqlen % nw == 0
    rows_per_w = seqlen // nw
    ch = min(_CHUNK_ROWS, rows_per_w)
    assert rows_per_w % ch == 0
    nch = rows_per_w // ch
    nbuf = min(_NBUF, nch)

    mesh = plsc.VectorSubcoreMesh(core_axis_name="c", subcore_axis_name="s")

    @functools.partial(
        pl.kernel,
        mesh=mesh,
        out_type=jax.ShapeDtypeStruct((seqlen, dim), pos_table.dtype),
        scratch_types=(
            [pltpu.VMEM((ch, dim), pos_table.dtype)] * nbuf
            + [pltpu.SemaphoreType.DMA] * (2 * nbuf)
        ),
    )
    def copy_k(table_hbm, out_hbm, *scratch):
        bufs = scratch[:nbuf]
        isems = scratch[nbuf : 2 * nbuf]
        osems = scratch[2 * nbuf :]
        wid = lax.axis_index("s") * nc + lax.axis_index("c")
        base = wid * rows_per_w

        in_h = [None] * nch
        out_h = [None] * nch
        for i in range(nbuf):
            in_h[i] = pltpu.async_copy(
                table_hbm.at[pl.ds(base + i * ch, ch)], bufs[i % nbuf], isems[i % nbuf]
            )
        for i in range(nch):
            b = i % nbuf
            in_h[i].wait()
            out_h[i] = pltpu.async_copy(
                bufs[b], out_hbm.at[pl.ds(base + i * ch, ch)], osems[b]
            )
            if i + nbuf < nch:
                out_h[i].wait()
                in_h[i + nbuf] = pltpu.async_copy(
                    table_hbm.at[pl.ds(base + (i + nbuf) * ch, ch)], bufs[b], isems[b]
                )
        for i in range(max(0, nch - nbuf), nch):
            out_h[i].wait()

    return copy_k(pos_table)
